# baseline (speedup 1.0000x reference)
import functools

import jax
import jax.numpy as jnp
from jax import lax
from jax.experimental import pallas as pl
from jax.experimental.pallas import tpu as pltpu
from jax.experimental.pallas import tpu_sc as plsc

NC, NS = 2, 16
NW = NC * NS
BATCH = 16384
D = 32
BPW = BATCH // NW
KU = 16
UV = 1000001
UTAIL0 = (UV // 128) * 128 - 128
UTAILS = UV - (UTAIL0 + 128)
TAILP = (UTAILS * D + 127) // 128
LV = 100001
LROWS = (LV * D + 127) // 128
LPAD = LROWS + (-LROWS % 8) + 8
LCH = 64
NLCH = BPW // LCH

_mesh = plsc.VectorSubcoreMesh(core_axis_name="c", subcore_axis_name="s")


def _splat(x):
    return jnp.broadcast_to(x, (16,))


@functools.partial(
    pl.kernel,
    out_type=(
        jax.ShapeDtypeStruct((D, BATCH), jnp.float32),
        jax.ShapeDtypeStruct((D, BATCH), jnp.float32),
    ),
    mesh=_mesh,
    compiler_params=pltpu.CompilerParams(use_tc_tiling_on_sc=True,
                                         needs_layout_passes=False),
    scratch_types=[
        pltpu.VMEM((BPW,), jnp.int32),
        pltpu.VMEM((BPW,), jnp.int32),
        pltpu.VMEM((BPW,), jnp.int32),
        pltpu.VMEM((KU, D, 128), jnp.float32),
        pltpu.VMEM((4, D, 128), jnp.float32),
        pltpu.VMEM((TAILP, 128), jnp.float32),
        pltpu.VMEM((2 * LCH, 128), jnp.float32),
        pltpu.VMEM((4, D, 128), jnp.float32),
    ] + [pltpu.SemaphoreType.DMA] * (KU + 2),
)
def _two_tower(uidx_hbm, lidx_hbm, utab_hbm, utail_hbm, llin_hbm,
               uout_hbm, lout_hbm,
               uidx_v, lidx_v, lrow_v,
               ublk, uasm, utail_v, lrows, lasm, *sems):
    usems = sems[:KU]
    lsems = sems[KU:]
    wid = lax.axis_index("s") * NC + lax.axis_index("c")
    base = pl.multiple_of(wid * BPW, BPW)
    iota16 = lax.iota(jnp.int32, 16)

    pltpu.sync_copy(uidx_hbm.at[pl.ds(base, BPW)], uidx_v)
    pltpu.sync_copy(lidx_hbm.at[pl.ds(base, BPW)], lidx_v)
    pltpu.sync_copy(utail_hbm, utail_v)

    for t in range(BPW // 16):
        lrow_v[pl.ds(t * 16, 16)] = lax.shift_right_logical(
            lidx_v[pl.ds(t * 16, 16)], 2)

    def fire_loc(i, sem):
        return pltpu.async_copy(
            llin_hbm.at[lrow_v.at[pl.ds(i * LCH, LCH)]],
            lrows.at[pl.ds((i % 2) * LCH, LCH)], sem)

    lcps = {i: fire_loc(i, lsems[i % 2]) for i in range(2)}

    def extract_user(slot, iu, c, b127):
        r = jnp.minimum(iu & 127, 127)
        tail = iu >= UTAIL0 + 128
        tflat = jnp.clip(iu - (UTAIL0 + 128), 0, UTAILS - 1) * D
        for h in range(2):
            j16 = iota16 + h * 16
            norm = plsc.load_gather(
                ublk, [jnp.full((16,), slot, jnp.int32), j16, _splat(r)])
            tf = tflat + j16
            tv = plsc.load_gather(
                utail_v, [lax.shift_right_logical(tf, 7), tf & 127])
            val = jnp.where(_splat(tail), tv, norm)
            plsc.store_scatter(
                uasm, [_splat(c), j16, _splat(b127)], val)

    def ubody(g, carry):
        vec = uidx_v[pl.ds(g * KU, KU)]
        cps = []
        for k in range(KU):
            iu = vec[k]
            off = pl.multiple_of(
                jnp.minimum(lax.shift_right_logical(iu, 7) * 128, UTAIL0),
                128)
            cps.append([
                pltpu.async_copy(
                    utab_hbm.at[pl.ds(8 * tr, 8), pl.ds(off, 128)],
                    ublk.at[k].at[pl.ds(8 * tr, 8), :], usems[k])
                for tr in range(4)
            ])
        for k in range(KU):
            b = g * KU + k
            for cp in cps[k]:
                cp.wait()
            extract_user(k, vec[k], lax.shift_right_logical(b, 7), b & 127)
        return carry

    lax.fori_loop(0, BPW // KU, ubody, 0, unroll=False)

    ucp = [
        pltpu.async_copy(uasm.at[c],
                         uout_hbm.at[:, pl.ds(base + c * 128, 128)], usems[c])
        for c in range(4)
    ]

    def extract_loc(row, off, c, b127):
        for h in range(2):
            j16 = iota16 + h * 16
            val = plsc.load_gather(lrows, [_splat(row), off + j16])
            plsc.store_scatter(
                lasm, [_splat(c), j16, _splat(b127)], val)

    for i in range(NLCH):
        lcps[i].wait()
        row0 = (i % 2) * LCH
        b0 = i * LCH

        def lbody(t, carry, row0=row0, b0=b0):
            vec = lidx_v[pl.ds(b0 + t * 16, 16)]
            for k in range(16):
                b = b0 + t * 16 + k
                extract_loc(row0 + t * 16 + k, (vec[k] & 3) * D,
                            lax.shift_right_logical(b, 7), b & 127)
            return carry

        lax.fori_loop(0, LCH // 16, lbody, 0, unroll=False)
        if i + 2 < NLCH:
            lcps[i + 2] = fire_loc(i + 2, lsems[i % 2])

    lcp = [
        pltpu.async_copy(lasm.at[c],
                         lout_hbm.at[:, pl.ds(base + c * 128, 128)],
                         usems[4 + c])
        for c in range(4)
    ]
    for cp in ucp + lcp:
        cp.wait()


def kernel(user_ids, gmap_ids, user_table, location_table):
    ut = jnp.swapaxes(user_table, 0, 1)
    tail = user_table[UTAIL0 + 128:, :].reshape(-1)
    tail = jnp.pad(tail, (0, TAILP * 128 - tail.shape[0]))
    utail = tail.reshape(TAILP, 128)
    lflat = location_table.reshape(-1)
    lpad = jnp.pad(lflat, (0, LPAD * 128 - lflat.shape[0]))
    llin = lpad.reshape(LPAD, 128)
    uo, lo = _two_tower(user_ids, gmap_ids, ut, utail, llin)
    return (jnp.swapaxes(uo, 0, 1), jnp.swapaxes(lo, 0, 1))


# speedup vs baseline: 1.0085x; 1.0085x over previous
import functools

import jax
import jax.numpy as jnp
from jax import lax
from jax.experimental import pallas as pl
from jax.experimental.pallas import tpu as pltpu
from jax.experimental.pallas import tpu_sc as plsc

NC, NS = 2, 16
NW = NC * NS
BATCH = 16384
D = 32
BPW = BATCH // NW
KU = 16
UV = 1000001
UTAIL0 = (UV // 128) * 128 - 128
UTAILS = UV - (UTAIL0 + 128)
TAILP = (UTAILS * D + 127) // 128
LV = 100001
LROWS = (LV * D + 127) // 128
LPAD = LROWS + (-LROWS % 8) + 8
LCH = 64
NLCH = BPW // LCH

_mesh = plsc.VectorSubcoreMesh(core_axis_name="c", subcore_axis_name="s")


def _splat(x):
    return jnp.broadcast_to(x, (16,))


@functools.partial(
    pl.kernel,
    out_type=(
        jax.ShapeDtypeStruct((D, BATCH), jnp.float32),
        jax.ShapeDtypeStruct((D, BATCH), jnp.float32),
    ),
    mesh=_mesh,
    compiler_params=pltpu.CompilerParams(use_tc_tiling_on_sc=True,
                                         needs_layout_passes=False,
                                         disable_bounds_checks=True,
                                         skip_device_barrier=True),
    scratch_types=[
        pltpu.VMEM((BPW,), jnp.int32),
        pltpu.VMEM((BPW,), jnp.int32),
        pltpu.VMEM((BPW,), jnp.int32),
        pltpu.VMEM((KU, D, 128), jnp.float32),
        pltpu.VMEM((4, D, 128), jnp.float32),
        pltpu.VMEM((TAILP, 128), jnp.float32),
        pltpu.VMEM((2 * LCH, 128), jnp.float32),
        pltpu.VMEM((4, D, 128), jnp.float32),
    ] + [pltpu.SemaphoreType.DMA] * (KU + 2),
)
def _two_tower(uidx_hbm, lidx_hbm, utab_hbm, utail_hbm, llin_hbm,
               uout_hbm, lout_hbm,
               uidx_v, lidx_v, lrow_v,
               ublk, uasm, utail_v, lrows, lasm, *sems):
    usems = sems[:KU]
    lsems = sems[KU:]
    wid = lax.axis_index("s") * NC + lax.axis_index("c")
    base = pl.multiple_of(wid * BPW, BPW)
    iota16 = lax.iota(jnp.int32, 16)

    pltpu.sync_copy(uidx_hbm.at[pl.ds(base, BPW)], uidx_v)
    pltpu.sync_copy(lidx_hbm.at[pl.ds(base, BPW)], lidx_v)
    pltpu.sync_copy(utail_hbm, utail_v)

    for t in range(BPW // 16):
        lrow_v[pl.ds(t * 16, 16)] = lax.shift_right_logical(
            lidx_v[pl.ds(t * 16, 16)], 2)

    def fire_loc(i, sem):
        return pltpu.async_copy(
            llin_hbm.at[lrow_v.at[pl.ds(i * LCH, LCH)]],
            lrows.at[pl.ds((i % 2) * LCH, LCH)], sem)

    lcps = {i: fire_loc(i, lsems[i % 2]) for i in range(2)}

    def extract_user(slot, iu, c, b127):
        r = jnp.minimum(iu & 127, 127)
        tail = iu >= UTAIL0 + 128
        tflat = jnp.clip(iu - (UTAIL0 + 128), 0, UTAILS - 1) * D
        for h in range(2):
            j16 = iota16 + h * 16
            norm = plsc.load_gather(
                ublk, [jnp.full((16,), slot, jnp.int32), j16, _splat(r)])
            tf = tflat + j16
            tv = plsc.load_gather(
                utail_v, [lax.shift_right_logical(tf, 7), tf & 127])
            val = jnp.where(_splat(tail), tv, norm)
            plsc.store_scatter(
                uasm, [_splat(c), j16, _splat(b127)], val)

    def ubody(g, carry):
        vec = uidx_v[pl.ds(g * KU, KU)]
        cps = []
        for k in range(KU):
            iu = vec[k]
            off = pl.multiple_of(
                jnp.minimum(lax.shift_right_logical(iu, 7) * 128, UTAIL0),
                128)
            cps.append(pltpu.async_copy(
                utab_hbm.at[:, pl.ds(off, 128)], ublk.at[k], usems[k]))
        for k in range(KU):
            b = g * KU + k
            cps[k].wait()
            extract_user(k, vec[k], lax.shift_right_logical(b, 7), b & 127)
        return carry

    lax.fori_loop(0, BPW // KU, ubody, 0, unroll=False)

    ucp = [
        pltpu.async_copy(uasm.at[c],
                         uout_hbm.at[:, pl.ds(base + c * 128, 128)], usems[c])
        for c in range(4)
    ]

    def extract_loc(row, off, c, b127):
        for h in range(2):
            j16 = iota16 + h * 16
            val = plsc.load_gather(lrows, [_splat(row), off + j16])
            plsc.store_scatter(
                lasm, [_splat(c), j16, _splat(b127)], val)

    for i in range(NLCH):
        lcps[i].wait()
        row0 = (i % 2) * LCH
        b0 = i * LCH

        def lbody(t, carry, row0=row0, b0=b0):
            vec = lidx_v[pl.ds(b0 + t * 16, 16)]
            for k in range(16):
                b = b0 + t * 16 + k
                extract_loc(row0 + t * 16 + k, (vec[k] & 3) * D,
                            lax.shift_right_logical(b, 7), b & 127)
            return carry

        lax.fori_loop(0, LCH // 16, lbody, 0, unroll=False)
        if i + 2 < NLCH:
            lcps[i + 2] = fire_loc(i + 2, lsems[i % 2])

    lcp = [
        pltpu.async_copy(lasm.at[c],
                         lout_hbm.at[:, pl.ds(base + c * 128, 128)],
                         usems[4 + c])
        for c in range(4)
    ]
    for cp in ucp + lcp:
        cp.wait()


def kernel(user_ids, gmap_ids, user_table, location_table):
    ut = jnp.swapaxes(user_table, 0, 1)
    tail = user_table[UTAIL0 + 128:, :].reshape(-1)
    tail = jnp.pad(tail, (0, TAILP * 128 - tail.shape[0]))
    utail = tail.reshape(TAILP, 128)
    lflat = location_table.reshape(-1)
    lpad = jnp.pad(lflat, (0, LPAD * 128 - lflat.shape[0]))
    llin = lpad.reshape(LPAD, 128)
    uo, lo = _two_tower(user_ids, gmap_ids, ut, utail, llin)
    return (jnp.swapaxes(uo, 0, 1), jnp.swapaxes(lo, 0, 1))


# speedup vs baseline: 1.1034x; 1.0941x over previous
import functools

import jax
import jax.numpy as jnp
from jax import lax
from jax.experimental import pallas as pl
from jax.experimental.pallas import tpu as pltpu
from jax.experimental.pallas import tpu_sc as plsc

NC, NS = 2, 16
NW = NC * NS
BATCH = 16384
D = 32
BPW = BATCH // NW
KU = 16
UV = 1000001
UTAIL0 = (UV // 128) * 128 - 128
UTAILS = UV - (UTAIL0 + 128)
TAILP = (UTAILS * D + 127) // 128
LV = 100001
LROWS = (LV * D + 127) // 128
LPAD = LROWS + (-LROWS % 8) + 8
LCH = 64
NLCH = BPW // LCH

_mesh = plsc.VectorSubcoreMesh(core_axis_name="c", subcore_axis_name="s")


def _splat(x):
    return jnp.broadcast_to(x, (16,))


@functools.partial(
    pl.kernel,
    out_type=(
        jax.ShapeDtypeStruct((D, BATCH), jnp.float32),
        jax.ShapeDtypeStruct((D, BATCH), jnp.float32),
    ),
    mesh=_mesh,
    compiler_params=pltpu.CompilerParams(use_tc_tiling_on_sc=True,
                                         needs_layout_passes=False,
                                         disable_bounds_checks=True,
                                         skip_device_barrier=True),
    scratch_types=[
        pltpu.VMEM((BPW,), jnp.int32),
        pltpu.VMEM((BPW,), jnp.int32),
        pltpu.VMEM((BPW,), jnp.int32),
        pltpu.VMEM((KU, D, 128), jnp.float32),
        pltpu.VMEM((4, D, 128), jnp.float32),
        pltpu.VMEM((TAILP, 128), jnp.float32),
        pltpu.VMEM((2 * LCH, 128), jnp.float32),
        pltpu.VMEM((4, D, 128), jnp.float32),
    ] + [pltpu.SemaphoreType.DMA] * (KU + 2),
)
def _two_tower(uidx_hbm, lidx_hbm, utab_hbm, utail_hbm, llin_hbm,
               uout_hbm, lout_hbm,
               uidx_v, lidx_v, lrow_v,
               ublk, uasm, utail_v, lrows, lasm, *sems):
    usems = sems[:KU]
    lsems = sems[KU:]
    wid = lax.axis_index("s") * NC + lax.axis_index("c")
    base = pl.multiple_of(wid * BPW, BPW)
    iota16 = lax.iota(jnp.int32, 16)

    pltpu.sync_copy(uidx_hbm.at[pl.ds(base, BPW)], uidx_v)
    pltpu.sync_copy(lidx_hbm.at[pl.ds(base, BPW)], lidx_v)
    pltpu.sync_copy(utail_hbm, utail_v)

    for t in range(BPW // 16):
        lrow_v[pl.ds(t * 16, 16)] = lax.shift_right_logical(
            lidx_v[pl.ds(t * 16, 16)], 2)

    def fire_loc(i, sem):
        return pltpu.async_copy(
            llin_hbm.at[lrow_v.at[pl.ds(i * LCH, LCH)]],
            lrows.at[pl.ds((i % 2) * LCH, LCH)], sem)

    lcps = {i: fire_loc(i, lsems[i % 2]) for i in range(2)}

    def extract_user(slot, iu, c, b127):
        r = jnp.minimum(iu & 127, 127)
        tail = iu >= UTAIL0 + 128
        tflat = jnp.clip(iu - (UTAIL0 + 128), 0, UTAILS - 1) * D
        for h in range(2):
            j16 = iota16 + h * 16
            norm = plsc.load_gather(
                ublk, [jnp.full((16,), slot, jnp.int32), j16, _splat(r)])
            tf = tflat + j16
            tv = plsc.load_gather(
                utail_v, [lax.shift_right_logical(tf, 7), tf & 127])
            val = jnp.where(_splat(tail), tv, norm)
            plsc.store_scatter(
                uasm, [_splat(c), j16, _splat(b127)], val)

    def fire_wave(vec, lane0, half):
        cps = []
        for k in range(8):
            iu = vec[lane0 + k]
            off = pl.multiple_of(
                jnp.minimum(lax.shift_right_logical(iu, 7) * 128, UTAIL0),
                128)
            slot = half * 8 + k
            cps.append(pltpu.async_copy(
                utab_hbm.at[:, pl.ds(off, 128)], ublk.at[slot], usems[slot]))
        return cps

    def drain_extract(vec, lane0, half, b0):
        for k in range(8):
            slot = half * 8 + k
            pltpu.make_async_copy(
                utab_hbm.at[:, pl.ds(pl.multiple_of(0, 128), 128)],
                ublk.at[slot], usems[slot]).wait()
            b = b0 + k
            extract_user(slot, vec[lane0 + k],
                         lax.shift_right_logical(b, 7), b & 127)

    NG = BPW // 16
    vec0 = uidx_v[pl.ds(0, 16)]
    fire_wave(vec0, 0, 0)

    def ubody(g, vec):
        fire_wave(vec, 8, 1)
        drain_extract(vec, 0, 0, g * 16)
        nxt = uidx_v[pl.ds(jnp.minimum((g + 1) * 16, BPW - 16), 16)]

        @pl.when(g < NG - 1)
        def _():
            fire_wave(nxt, 0, 0)

        drain_extract(vec, 8, 1, g * 16 + 8)
        return nxt

    lax.fori_loop(0, NG, ubody, vec0, unroll=False)

    ucp = [
        pltpu.async_copy(uasm.at[c],
                         uout_hbm.at[:, pl.ds(base + c * 128, 128)], usems[c])
        for c in range(4)
    ]

    def extract_loc(row, off, c, b127):
        for h in range(2):
            j16 = iota16 + h * 16
            val = plsc.load_gather(lrows, [_splat(row), off + j16])
            plsc.store_scatter(
                lasm, [_splat(c), j16, _splat(b127)], val)

    for i in range(NLCH):
        lcps[i].wait()
        row0 = (i % 2) * LCH
        b0 = i * LCH

        def lbody(t, carry, row0=row0, b0=b0):
            vec = lidx_v[pl.ds(b0 + t * 16, 16)]
            for k in range(16):
                b = b0 + t * 16 + k
                extract_loc(row0 + t * 16 + k, (vec[k] & 3) * D,
                            lax.shift_right_logical(b, 7), b & 127)
            return carry

        lax.fori_loop(0, LCH // 16, lbody, 0, unroll=False)
        if i + 2 < NLCH:
            lcps[i + 2] = fire_loc(i + 2, lsems[i % 2])

    lcp = [
        pltpu.async_copy(lasm.at[c],
                         lout_hbm.at[:, pl.ds(base + c * 128, 128)],
                         usems[4 + c])
        for c in range(4)
    ]
    for cp in ucp + lcp:
        cp.wait()


def kernel(user_ids, gmap_ids, user_table, location_table):
    ut = jnp.swapaxes(user_table, 0, 1)
    tail = user_table[UTAIL0 + 128:, :].reshape(-1)
    tail = jnp.pad(tail, (0, TAILP * 128 - tail.shape[0]))
    utail = tail.reshape(TAILP, 128)
    lflat = location_table.reshape(-1)
    lpad = jnp.pad(lflat, (0, LPAD * 128 - lflat.shape[0]))
    llin = lpad.reshape(LPAD, 128)
    uo, lo = _two_tower(user_ids, gmap_ids, ut, utail, llin)
    return (jnp.swapaxes(uo, 0, 1), jnp.swapaxes(lo, 0, 1))
